# Initial kernel scaffold; baseline (speedup 1.0000x reference)
import jax
import jax.numpy as jnp
from jax.experimental import pallas as pl


def kernel(features, edge_index, W1, b1, g1, bt1, W2, b2, g2, bt2, W3, b3, g3, bt3):
    raise NotImplementedError("write your pallas kernel here")


# speedup vs baseline: 17.4981x; 17.4981x over previous
import functools

import jax
import jax.numpy as jnp
from jax import lax
from jax.experimental import pallas as pl
from jax.experimental.pallas import tpu as pltpu
from jax.experimental.pallas import tpu_sc as plsc

_N = 10000
_D = 128
_E = 320000
_NC = 2
_NS = 16
_NW = _NC * _NS
_CH = 128
_G = -(-_E // (_NW * _CH))
_EPAD = _NW * _G * _CH
_NPAD = 10240
_RPS = _NPAD // _NS
_DW = 16

_mesh = plsc.VectorSubcoreMesh(
    core_axis_name="c", subcore_axis_name="s", num_cores=_NC, num_subcores=_NS
)


@functools.partial(
    pl.kernel,
    out_type=jax.ShapeDtypeStruct((_NC, _NPAD, _DW), jnp.float32),
    mesh=_mesh,
    scratch_types=[
        pltpu.VMEM((_G, _CH), jnp.int32),
        pltpu.VMEM((_CH, _DW), jnp.float32),
        pltpu.VMEM((16, _DW), jnp.float32),
        pltpu.VMEM_SHARED((_NPAD, _DW), jnp.float32),
    ],
)
def _deg_kernel(dst_hbm, out_hbm, dst_v, ones_v, zb_v, acc_sh):
    c = lax.axis_index("c")
    s = lax.axis_index("s")
    w = c * _NS + s
    one16 = jnp.full((16,), 1.0, jnp.float32)
    zero16 = jnp.zeros((16,), jnp.float32)
    for r in range(_CH):
        ones_v[r, :] = one16
    for r in range(16):
        zb_v[r, :] = zero16
    base = s * _RPS

    def zinit(k, carry):
        pltpu.sync_copy(zb_v, acc_sh.at[pl.ds(base + k * 16, 16)])
        return carry

    lax.fori_loop(0, _RPS // 16, zinit, 0)
    pltpu.sync_copy(dst_hbm.at[w], dst_v)
    plsc.subcore_barrier()

    def body(g, carry):
        pltpu.sync_copy(ones_v, acc_sh.at[dst_v.at[g]], add=True)
        return carry

    lax.fori_loop(0, _G, body, 0)
    plsc.subcore_barrier()
    pltpu.sync_copy(acc_sh.at[pl.ds(base, _RPS)], out_hbm.at[c, pl.ds(base, _RPS)])


@functools.partial(
    pl.kernel,
    out_type=jax.ShapeDtypeStruct((_NC, _NPAD, _D), jnp.float32),
    mesh=_mesh,
    scratch_types=[
        pltpu.VMEM((_G, _CH), jnp.int32),
        pltpu.VMEM((_G, _CH), jnp.int32),
        pltpu.VMEM((_CH, _D), jnp.float32),
        pltpu.VMEM((16, _D), jnp.float32),
        pltpu.VMEM_SHARED((_NPAD, _D), jnp.float32),
        pltpu.SemaphoreType.DMA,
    ],
)
def _gather_scatter_kernel(xs_hbm, src_hbm, dst_hbm, out_hbm,
                           src_v, dst_v, rows_v, zb_v, acc_sh, sem):
    c = lax.axis_index("c")
    s = lax.axis_index("s")
    w = c * _NS + s
    zero16 = jnp.zeros((16,), jnp.float32)
    for r in range(16):
        for j in range(_D // 16):
            zb_v[r, pl.ds(j * 16, 16)] = zero16
    base = s * _RPS

    def zinit(k, carry):
        pltpu.sync_copy(zb_v, acc_sh.at[pl.ds(base + k * 16, 16)])
        return carry

    lax.fori_loop(0, _RPS // 16, zinit, 0)
    pltpu.sync_copy(src_hbm.at[w], src_v)
    pltpu.sync_copy(dst_hbm.at[w], dst_v)
    plsc.subcore_barrier()

    def body(g, carry):
        pltpu.async_copy(xs_hbm.at[src_v.at[g]], rows_v, sem).wait()
        pltpu.sync_copy(rows_v, acc_sh.at[dst_v.at[g]], add=True)
        return carry

    lax.fori_loop(0, _G, body, 0)
    plsc.subcore_barrier()
    pltpu.sync_copy(acc_sh.at[pl.ds(base, _RPS)], out_hbm.at[c, pl.ds(base, _RPS)])


_R = 512


def _pre_body(f_ref, d0_ref, d1_ref, xs_ref):
    dinv = lax.rsqrt(d0_ref[...] + d1_ref[...] + 1.0)
    xs_ref[...] = f_ref[...] * dinv


_pre_kernel = pl.pallas_call(
    _pre_body,
    grid=(_NPAD // _R,),
    in_specs=[
        pl.BlockSpec((_R, _D), lambda b: (b, 0)),
        pl.BlockSpec((_R, 1), lambda b: (b, 0)),
        pl.BlockSpec((_R, 1), lambda b: (b, 0)),
    ],
    out_specs=pl.BlockSpec((_R, _D), lambda b: (b, 0)),
    out_shape=jax.ShapeDtypeStruct((_NPAD, _D), jnp.float32),
)


def _layer_body(p0_ref, p1_ref, xs_ref, d0_ref, d1_ref, w_ref, b_ref,
                gm_ref, bt_ref, o_ref, xsn_ref):
    dinv = lax.rsqrt(d0_ref[...] + d1_ref[...] + 1.0)
    a = (p0_ref[...] + p1_ref[...] + xs_ref[...]) * dinv
    t = jnp.dot(a, w_ref[...], preferred_element_type=jnp.float32) + b_ref[...]
    m = jnp.mean(t, axis=-1, keepdims=True)
    d = t - m
    v = jnp.mean(d * d, axis=-1, keepdims=True)
    o = jnp.maximum(d * lax.rsqrt(v + 1e-5) * gm_ref[...] + bt_ref[...], 0.0)
    o_ref[...] = o
    xsn_ref[...] = o * dinv


def _final_body(p0_ref, p1_ref, xs_ref, d0_ref, d1_ref, w_ref, b_ref,
                gm_ref, bt_ref, o1_ref, o2_ref, out_ref):
    dinv = lax.rsqrt(d0_ref[...] + d1_ref[...] + 1.0)
    a = (p0_ref[...] + p1_ref[...] + xs_ref[...]) * dinv
    t = jnp.dot(a, w_ref[...], preferred_element_type=jnp.float32) + b_ref[...]
    m = jnp.mean(t, axis=-1, keepdims=True)
    d = t - m
    v = jnp.mean(d * d, axis=-1, keepdims=True)
    y = d * lax.rsqrt(v + 1e-5) * gm_ref[...] + bt_ref[...]
    out_ref[...] = jnp.maximum(jnp.maximum(o1_ref[...], o2_ref[...]), y)


_row_spec = pl.BlockSpec((_R, _D), lambda b: (b, 0))
_col_spec = pl.BlockSpec((_R, 1), lambda b: (b, 0))
_w_spec = pl.BlockSpec((_D, _D), lambda b: (0, 0))
_vec_spec = pl.BlockSpec((1, _D), lambda b: (0, 0))

_layer_kernel = pl.pallas_call(
    _layer_body,
    grid=(_NPAD // _R,),
    in_specs=[_row_spec, _row_spec, _row_spec, _col_spec, _col_spec,
              _w_spec, _vec_spec, _vec_spec, _vec_spec],
    out_specs=[_row_spec, _row_spec],
    out_shape=[jax.ShapeDtypeStruct((_NPAD, _D), jnp.float32),
               jax.ShapeDtypeStruct((_NPAD, _D), jnp.float32)],
)

_final_kernel = pl.pallas_call(
    _final_body,
    grid=(_NPAD // _R,),
    in_specs=[_row_spec, _row_spec, _row_spec, _col_spec, _col_spec,
              _w_spec, _vec_spec, _vec_spec, _vec_spec, _row_spec, _row_spec],
    out_specs=_row_spec,
    out_shape=jax.ShapeDtypeStruct((_NPAD, _D), jnp.float32),
)


def kernel(features, edge_index, W1, b1, g1, bt1, W2, b2, g2, bt2,
           W3, b3, g3, bt3):
    npad = _NPAD - _N
    epad = _EPAD - _E
    fill = _N + (jnp.arange(epad, dtype=jnp.int32) % npad)
    srcp = jnp.concatenate([edge_index[0], fill]).reshape(_NW, _G, _CH)
    dstp = jnp.concatenate([edge_index[1], fill]).reshape(_NW, _G, _CH)
    fpad = jnp.concatenate(
        [features, jnp.zeros((npad, _D), jnp.float32)], axis=0)

    degp = _deg_kernel(dstp)
    d0 = degp[0, :, :1]
    d1 = degp[1, :, :1]

    def row(x):
        return x.reshape(1, _D)

    xs1 = _pre_kernel(fpad, d0, d1)
    p = _gather_scatter_kernel(xs1, srcp, dstp)
    o1, xs2 = _layer_kernel(p[0], p[1], xs1, d0, d1, W1, row(b1), row(g1),
                            row(bt1))
    p = _gather_scatter_kernel(xs2, srcp, dstp)
    o2, xs3 = _layer_kernel(p[0], p[1], xs2, d0, d1, W2, row(b2), row(g2),
                            row(bt2))
    p = _gather_scatter_kernel(xs3, srcp, dstp)
    out = _final_kernel(p[0], p[1], xs3, d0, d1, W3, row(b3), row(g3),
                        row(bt3), o1, o2)
    return out[:_N]


# speedup vs baseline: 22.8629x; 1.3066x over previous
import functools

import jax
import jax.numpy as jnp
from jax import lax
from jax.experimental import pallas as pl
from jax.experimental.pallas import tpu as pltpu
from jax.experimental.pallas import tpu_sc as plsc

_N = 10000
_D = 128
_E = 320000
_NC = 2
_NS = 16
_NW = _NC * _NS
_CH = 128
_IB = 16
_G = 80
_EPAD = _NW * _G * _CH
_NPAD = 10240
_RPS = _NPAD // _NS
_DW = 16

_mesh = plsc.VectorSubcoreMesh(
    core_axis_name="c", subcore_axis_name="s", num_cores=_NC, num_subcores=_NS
)


@functools.partial(
    pl.kernel,
    out_type=jax.ShapeDtypeStruct((_NC, _NPAD, _DW), jnp.float32),
    mesh=_mesh,
    scratch_types=[
        pltpu.VMEM((_G, _CH), jnp.int32),
        pltpu.VMEM((_CH, _DW), jnp.float32),
        pltpu.VMEM((16, _DW), jnp.float32),
        pltpu.VMEM_SHARED((_NPAD, _DW), jnp.float32),
    ],
)
def _deg_kernel(dst_hbm, out_hbm, dst_v, ones_v, zb_v, acc_sh):
    c = lax.axis_index("c")
    s = lax.axis_index("s")
    w = c * _NS + s
    one16 = jnp.full((16,), 1.0, jnp.float32)
    zero16 = jnp.zeros((16,), jnp.float32)
    for r in range(_CH):
        ones_v[r, :] = one16
    for r in range(16):
        zb_v[r, :] = zero16
    base = s * _RPS

    def zinit(k, carry):
        pltpu.sync_copy(zb_v, acc_sh.at[pl.ds(base + k * 16, 16)])
        return carry

    lax.fori_loop(0, _RPS // 16, zinit, 0)
    pltpu.sync_copy(dst_hbm.at[w], dst_v)
    plsc.subcore_barrier()

    def body(g, carry):
        pltpu.sync_copy(ones_v, acc_sh.at[dst_v.at[g]], add=True)
        return carry

    lax.fori_loop(0, _G, body, 0)
    plsc.subcore_barrier()
    pltpu.sync_copy(acc_sh.at[pl.ds(base, _RPS)], out_hbm.at[c, pl.ds(base, _RPS)])


@functools.partial(
    pl.kernel,
    out_type=jax.ShapeDtypeStruct((_NC, _NPAD, _D), jnp.float32),
    mesh=_mesh,
    scratch_types=[
        pltpu.VMEM((_IB, _CH), jnp.int32),
        pltpu.VMEM((_IB, _CH), jnp.int32),
        pltpu.VMEM((2, _CH, _D), jnp.float32),
        pltpu.VMEM((8, _D), jnp.float32),
        pltpu.VMEM_SHARED((_NPAD, _D), jnp.float32),
        pltpu.SemaphoreType.DMA((2,)),
    ],
)
def _gather_scatter_kernel(xs_hbm, src_hbm, dst_hbm, out_hbm,
                           src_v, dst_v, rows_v, zb_v, acc_sh, sems):
    c = lax.axis_index("c")
    s = lax.axis_index("s")
    w = c * _NS + s
    zero16 = jnp.zeros((16,), jnp.float32)
    for r in range(8):
        for j in range(_D // 16):
            zb_v[r, pl.ds(j * 16, 16)] = zero16
    base = s * _RPS

    def zinit(k, carry):
        pltpu.sync_copy(zb_v, acc_sh.at[pl.ds(base + k * 8, 8)])
        return carry

    lax.fori_loop(0, _RPS // 8, zinit, 0)
    plsc.subcore_barrier()

    def phase(p, carry):
        pltpu.sync_copy(src_hbm.at[w, pl.ds(p * _IB, _IB)], src_v)
        pltpu.sync_copy(dst_hbm.at[w, pl.ds(p * _IB, _IB)], dst_v)

        descs = [
            pltpu.async_copy(xs_hbm.at[src_v.at[0]], rows_v.at[0], sems.at[0]),
            None,
        ]
        for g in range(_IB):
            b = g % 2
            nb = (g + 1) % 2
            if g + 1 < _IB:
                descs[nb] = pltpu.async_copy(xs_hbm.at[src_v.at[g + 1]],
                                             rows_v.at[nb], sems.at[nb])
            descs[b].wait()
            pltpu.sync_copy(rows_v.at[b], acc_sh.at[dst_v.at[g]], add=True)
        return carry

    lax.fori_loop(0, _G // _IB, phase, 0)
    plsc.subcore_barrier()
    pltpu.sync_copy(acc_sh.at[pl.ds(base, _RPS)], out_hbm.at[c, pl.ds(base, _RPS)])


_R = 512


def _pre_body(f_ref, d0_ref, d1_ref, xs_ref):
    dinv = lax.rsqrt(d0_ref[...] + d1_ref[...] + 1.0)
    xs_ref[...] = f_ref[...] * dinv


_pre_kernel = pl.pallas_call(
    _pre_body,
    grid=(_NPAD // _R,),
    in_specs=[
        pl.BlockSpec((_R, _D), lambda b: (b, 0)),
        pl.BlockSpec((_R, 1), lambda b: (b, 0)),
        pl.BlockSpec((_R, 1), lambda b: (b, 0)),
    ],
    out_specs=pl.BlockSpec((_R, _D), lambda b: (b, 0)),
    out_shape=jax.ShapeDtypeStruct((_NPAD, _D), jnp.float32),
)


def _layer_body(p0_ref, p1_ref, xs_ref, d0_ref, d1_ref, w_ref, b_ref,
                gm_ref, bt_ref, o_ref, xsn_ref):
    dinv = lax.rsqrt(d0_ref[...] + d1_ref[...] + 1.0)
    a = (p0_ref[...] + p1_ref[...] + xs_ref[...]) * dinv
    t = jnp.dot(a, w_ref[...], preferred_element_type=jnp.float32) + b_ref[...]
    m = jnp.mean(t, axis=-1, keepdims=True)
    d = t - m
    v = jnp.mean(d * d, axis=-1, keepdims=True)
    o = jnp.maximum(d * lax.rsqrt(v + 1e-5) * gm_ref[...] + bt_ref[...], 0.0)
    o_ref[...] = o
    xsn_ref[...] = o * dinv


def _final_body(p0_ref, p1_ref, xs_ref, d0_ref, d1_ref, w_ref, b_ref,
                gm_ref, bt_ref, o1_ref, o2_ref, out_ref):
    dinv = lax.rsqrt(d0_ref[...] + d1_ref[...] + 1.0)
    a = (p0_ref[...] + p1_ref[...] + xs_ref[...]) * dinv
    t = jnp.dot(a, w_ref[...], preferred_element_type=jnp.float32) + b_ref[...]
    m = jnp.mean(t, axis=-1, keepdims=True)
    d = t - m
    v = jnp.mean(d * d, axis=-1, keepdims=True)
    y = d * lax.rsqrt(v + 1e-5) * gm_ref[...] + bt_ref[...]
    out_ref[...] = jnp.maximum(jnp.maximum(o1_ref[...], o2_ref[...]), y)


_row_spec = pl.BlockSpec((_R, _D), lambda b: (b, 0))
_col_spec = pl.BlockSpec((_R, 1), lambda b: (b, 0))
_w_spec = pl.BlockSpec((_D, _D), lambda b: (0, 0))
_vec_spec = pl.BlockSpec((1, _D), lambda b: (0, 0))

_layer_kernel = pl.pallas_call(
    _layer_body,
    grid=(_NPAD // _R,),
    in_specs=[_row_spec, _row_spec, _row_spec, _col_spec, _col_spec,
              _w_spec, _vec_spec, _vec_spec, _vec_spec],
    out_specs=[_row_spec, _row_spec],
    out_shape=[jax.ShapeDtypeStruct((_NPAD, _D), jnp.float32),
               jax.ShapeDtypeStruct((_NPAD, _D), jnp.float32)],
)

_final_kernel = pl.pallas_call(
    _final_body,
    grid=(_NPAD // _R,),
    in_specs=[_row_spec, _row_spec, _row_spec, _col_spec, _col_spec,
              _w_spec, _vec_spec, _vec_spec, _vec_spec, _row_spec, _row_spec],
    out_specs=_row_spec,
    out_shape=jax.ShapeDtypeStruct((_NPAD, _D), jnp.float32),
)


def kernel(features, edge_index, W1, b1, g1, bt1, W2, b2, g2, bt2,
           W3, b3, g3, bt3):
    npad = _NPAD - _N
    epad = _EPAD - _E
    fill = _N + (jnp.arange(epad, dtype=jnp.int32) % npad)
    srcp = jnp.concatenate([edge_index[0], fill]).reshape(_NW, _G, _CH)
    dstp = jnp.concatenate([edge_index[1], fill]).reshape(_NW, _G, _CH)
    fpad = jnp.concatenate(
        [features, jnp.zeros((npad, _D), jnp.float32)], axis=0)

    degp = _deg_kernel(dstp)
    d0 = degp[0, :, :1]
    d1 = degp[1, :, :1]

    def row(x):
        return x.reshape(1, _D)

    xs1 = _pre_kernel(fpad, d0, d1)
    p = _gather_scatter_kernel(xs1, srcp, dstp)
    o1, xs2 = _layer_kernel(p[0], p[1], xs1, d0, d1, W1, row(b1), row(g1),
                            row(bt1))
    p = _gather_scatter_kernel(xs2, srcp, dstp)
    o2, xs3 = _layer_kernel(p[0], p[1], xs2, d0, d1, W2, row(b2), row(g2),
                            row(bt2))
    p = _gather_scatter_kernel(xs3, srcp, dstp)
    out = _final_kernel(p[0], p[1], xs3, d0, d1, W3, row(b3), row(g3),
                        row(bt3), o1, o2)
    return out[:_N]


# speedup vs baseline: 24.1038x; 1.0543x over previous
import functools

import jax
import jax.numpy as jnp
from jax import lax
from jax.experimental import pallas as pl
from jax.experimental.pallas import tpu as pltpu
from jax.experimental.pallas import tpu_sc as plsc

_N = 10000
_D = 128
_E = 320000
_NC = 2
_NS = 16
_NW = _NC * _NS
_CH = 128
_IB = 40
_G = 80
_EPAD = _NW * _G * _CH
_NPAD = 10240
_RPS = _NPAD // _NS
_DW = 16

_mesh = plsc.VectorSubcoreMesh(
    core_axis_name="c", subcore_axis_name="s", num_cores=_NC, num_subcores=_NS
)


@functools.partial(
    pl.kernel,
    out_type=jax.ShapeDtypeStruct((_NC, _NPAD, _DW), jnp.float32),
    mesh=_mesh,
    scratch_types=[
        pltpu.VMEM((_G, _CH), jnp.int32),
        pltpu.VMEM((_CH, _DW), jnp.float32),
        pltpu.VMEM((16, _DW), jnp.float32),
        pltpu.VMEM_SHARED((_NPAD, _DW), jnp.float32),
    ],
)
def _deg_kernel(dst_hbm, out_hbm, dst_v, ones_v, zb_v, acc_sh):
    c = lax.axis_index("c")
    s = lax.axis_index("s")
    w = c * _NS + s
    one16 = jnp.full((16,), 1.0, jnp.float32)
    zero16 = jnp.zeros((16,), jnp.float32)
    for r in range(_CH):
        ones_v[r, :] = one16
    for r in range(16):
        zb_v[r, :] = zero16
    base = s * _RPS

    def zinit(k, carry):
        pltpu.sync_copy(zb_v, acc_sh.at[pl.ds(base + k * 16, 16)])
        return carry

    lax.fori_loop(0, _RPS // 16, zinit, 0)
    pltpu.sync_copy(dst_hbm.at[w], dst_v)
    plsc.subcore_barrier()

    def body(g, carry):
        pltpu.sync_copy(ones_v, acc_sh.at[dst_v.at[g]], add=True)
        return carry

    lax.fori_loop(0, _G, body, 0)
    plsc.subcore_barrier()
    pltpu.sync_copy(acc_sh.at[pl.ds(base, _RPS)], out_hbm.at[c, pl.ds(base, _RPS)])


@functools.partial(
    pl.kernel,
    out_type=jax.ShapeDtypeStruct((_NC, _NPAD, _D), jnp.float32),
    mesh=_mesh,
    scratch_types=[
        pltpu.VMEM((_IB, _CH), jnp.int32),
        pltpu.VMEM((_IB, _CH), jnp.int32),
        pltpu.VMEM((2, _CH, _D), jnp.float32),
        pltpu.VMEM((8, _D), jnp.float32),
        pltpu.VMEM_SHARED((_NPAD, _D), jnp.float32),
        pltpu.SemaphoreType.DMA((2,)),
    ],
)
def _gather_scatter_kernel(xs_hbm, src_hbm, dst_hbm, out_hbm,
                           src_v, dst_v, rows_v, zb_v, acc_sh, sems):
    c = lax.axis_index("c")
    s = lax.axis_index("s")
    w = c * _NS + s
    zero16 = jnp.zeros((16,), jnp.float32)
    for r in range(8):
        for j in range(_D // 16):
            zb_v[r, pl.ds(j * 16, 16)] = zero16
    base = s * _RPS

    def zinit(k, carry):
        pltpu.sync_copy(zb_v, acc_sh.at[pl.ds(base + k * 8, 8)])
        return carry

    lax.fori_loop(0, _RPS // 8, zinit, 0)
    plsc.subcore_barrier()

    def phase(p, carry):
        pltpu.sync_copy(src_hbm.at[w, pl.ds(p * _IB, _IB)], src_v)
        pltpu.sync_copy(dst_hbm.at[w, pl.ds(p * _IB, _IB)], dst_v)

        descs = [
            pltpu.async_copy(xs_hbm.at[src_v.at[0]], rows_v.at[0], sems.at[0]),
            None,
        ]
        for g in range(_IB):
            b = g % 2
            nb = (g + 1) % 2
            if g + 1 < _IB:
                descs[nb] = pltpu.async_copy(xs_hbm.at[src_v.at[g + 1]],
                                             rows_v.at[nb], sems.at[nb])
            descs[b].wait()
            pltpu.sync_copy(rows_v.at[b], acc_sh.at[dst_v.at[g]], add=True)
        return carry

    lax.fori_loop(0, _G // _IB, phase, 0)
    plsc.subcore_barrier()
    pltpu.sync_copy(acc_sh.at[pl.ds(base, _RPS)], out_hbm.at[c, pl.ds(base, _RPS)])


_R = 512


def _pre_body(f_ref, d0_ref, d1_ref, xs_ref):
    dinv = lax.rsqrt(d0_ref[...] + d1_ref[...] + 1.0)
    xs_ref[...] = f_ref[...] * dinv


_pre_kernel = pl.pallas_call(
    _pre_body,
    grid=(_NPAD // _R,),
    in_specs=[
        pl.BlockSpec((_R, _D), lambda b: (b, 0)),
        pl.BlockSpec((_R, 1), lambda b: (b, 0)),
        pl.BlockSpec((_R, 1), lambda b: (b, 0)),
    ],
    out_specs=pl.BlockSpec((_R, _D), lambda b: (b, 0)),
    out_shape=jax.ShapeDtypeStruct((_NPAD, _D), jnp.float32),
)


def _layer_body(p0_ref, p1_ref, xs_ref, d0_ref, d1_ref, w_ref, b_ref,
                gm_ref, bt_ref, o_ref, xsn_ref):
    dinv = lax.rsqrt(d0_ref[...] + d1_ref[...] + 1.0)
    a = (p0_ref[...] + p1_ref[...] + xs_ref[...]) * dinv
    t = jnp.dot(a, w_ref[...], preferred_element_type=jnp.float32) + b_ref[...]
    m = jnp.mean(t, axis=-1, keepdims=True)
    d = t - m
    v = jnp.mean(d * d, axis=-1, keepdims=True)
    o = jnp.maximum(d * lax.rsqrt(v + 1e-5) * gm_ref[...] + bt_ref[...], 0.0)
    o_ref[...] = o
    xsn_ref[...] = o * dinv


def _final_body(p0_ref, p1_ref, xs_ref, d0_ref, d1_ref, w_ref, b_ref,
                gm_ref, bt_ref, o1_ref, o2_ref, out_ref):
    dinv = lax.rsqrt(d0_ref[...] + d1_ref[...] + 1.0)
    a = (p0_ref[...] + p1_ref[...] + xs_ref[...]) * dinv
    t = jnp.dot(a, w_ref[...], preferred_element_type=jnp.float32) + b_ref[...]
    m = jnp.mean(t, axis=-1, keepdims=True)
    d = t - m
    v = jnp.mean(d * d, axis=-1, keepdims=True)
    y = d * lax.rsqrt(v + 1e-5) * gm_ref[...] + bt_ref[...]
    out_ref[...] = jnp.maximum(jnp.maximum(o1_ref[...], o2_ref[...]), y)


_row_spec = pl.BlockSpec((_R, _D), lambda b: (b, 0))
_col_spec = pl.BlockSpec((_R, 1), lambda b: (b, 0))
_w_spec = pl.BlockSpec((_D, _D), lambda b: (0, 0))
_vec_spec = pl.BlockSpec((1, _D), lambda b: (0, 0))

_layer_kernel = pl.pallas_call(
    _layer_body,
    grid=(_NPAD // _R,),
    in_specs=[_row_spec, _row_spec, _row_spec, _col_spec, _col_spec,
              _w_spec, _vec_spec, _vec_spec, _vec_spec],
    out_specs=[_row_spec, _row_spec],
    out_shape=[jax.ShapeDtypeStruct((_NPAD, _D), jnp.float32),
               jax.ShapeDtypeStruct((_NPAD, _D), jnp.float32)],
)

_final_kernel = pl.pallas_call(
    _final_body,
    grid=(_NPAD // _R,),
    in_specs=[_row_spec, _row_spec, _row_spec, _col_spec, _col_spec,
              _w_spec, _vec_spec, _vec_spec, _vec_spec, _row_spec, _row_spec],
    out_specs=_row_spec,
    out_shape=jax.ShapeDtypeStruct((_NPAD, _D), jnp.float32),
)


def kernel(features, edge_index, W1, b1, g1, bt1, W2, b2, g2, bt2,
           W3, b3, g3, bt3):
    npad = _NPAD - _N
    epad = _EPAD - _E
    fill = _N + (jnp.arange(epad, dtype=jnp.int32) % npad)
    srcp = jnp.concatenate([edge_index[0], fill]).reshape(_NW, _G, _CH)
    dstp = jnp.concatenate([edge_index[1], fill]).reshape(_NW, _G, _CH)
    fpad = jnp.concatenate(
        [features, jnp.zeros((npad, _D), jnp.float32)], axis=0)

    degp = _deg_kernel(dstp)
    d0 = degp[0, :, :1]
    d1 = degp[1, :, :1]

    def row(x):
        return x.reshape(1, _D)

    xs1 = _pre_kernel(fpad, d0, d1)
    p = _gather_scatter_kernel(xs1, srcp, dstp)
    o1, xs2 = _layer_kernel(p[0], p[1], xs1, d0, d1, W1, row(b1), row(g1),
                            row(bt1))
    p = _gather_scatter_kernel(xs2, srcp, dstp)
    o2, xs3 = _layer_kernel(p[0], p[1], xs2, d0, d1, W2, row(b2), row(g2),
                            row(bt2))
    p = _gather_scatter_kernel(xs3, srcp, dstp)
    out = _final_kernel(p[0], p[1], xs3, d0, d1, W3, row(b3), row(g3),
                        row(bt3), o1, o2)
    return out[:_N]
